# baseline (speedup 1.0000x reference)
import functools

import jax
import jax.numpy as jnp
from jax import lax
from jax.experimental import pallas as pl
from jax.experimental.pallas import tpu as pltpu
from jax.experimental.pallas import tpu_sc as plsc

B, T, D = 4, 4096, 4096
E = 64
TOPK = 8
N_SHARED = 2
N_OUT = N_SHARED + TOPK
ROUTE_SCALE = 2.5
N_TOK = B * T

TBLK = 512


def _keys_block(x_ref, wt_ref, bias_ref, k_ref):
    s = jnp.dot(x_ref[...], wt_ref[...], preferred_element_type=jnp.float32)
    m = jnp.max(s, axis=1, keepdims=True)
    p = jnp.exp(s - m)
    probs = p / jnp.sum(p, axis=1, keepdims=True)
    k_ref[...] = probs + bias_ref[...]


def _keys(x2d, wt, bias2d, n, blk0):
    return pl.pallas_call(
        _keys_block,
        grid=(n // TBLK,),
        in_specs=[
            pl.BlockSpec((TBLK, D), lambda i: (i + blk0, 0)),
            pl.BlockSpec((D, E), lambda i: (0, 0)),
            pl.BlockSpec((1, E), lambda i: (0, 0)),
        ],
        out_specs=pl.BlockSpec((TBLK, E), lambda i: (i, 0)),
        out_shape=jax.ShapeDtypeStruct((n, E), jnp.float32),
    )(x2d, wt, bias2d)


_L = 16
NW = 32
NCHUNK = 2
CTOK = N_TOK // NCHUNK
CHUNK = CTOK // NW
NGRP = CHUNK // _L


def _topk_body(keys_hbm, bias_hbm, w_hbm, i_hbm, keys_v, bias_v, w_v, i_v, nc):
    wid = lax.axis_index("s") * nc + lax.axis_index("c")
    pltpu.sync_copy(keys_hbm.at[pl.ds(wid * CHUNK * E, CHUNK * E)], keys_v)
    pltpu.sync_copy(bias_hbm, bias_v)

    def group_body(g, gcarry):
        tok = lax.iota(jnp.int32, _L) + g * _L
        flat0 = tok * E
        neg = jnp.full((_L,), -jnp.inf, jnp.float32)
        zero = jnp.zeros((_L,), jnp.int32)
        carry0 = (neg,) * TOPK + (zero,) * TOPK

        def exp_body(e, carry):
            ks = carry[:TOPK]
            ix = carry[TOPK:]
            e_spl = jnp.full((_L,), e, jnp.int32)
            v = plsc.load_gather(keys_v, [flat0 + e])
            c = [v > ks[j] for j in range(TOPK)]
            nk = [jnp.where(c[0], v, ks[0])]
            ni = [jnp.where(c[0], e_spl, ix[0])]
            for j in range(1, TOPK):
                nk.append(jnp.where(c[j], jnp.where(c[j - 1], ks[j - 1], v), ks[j]))
                ni.append(jnp.where(c[j], jnp.where(c[j - 1], ix[j - 1], e_spl), ix[j]))
            return tuple(nk) + tuple(ni)

        res = lax.fori_loop(0, E, exp_body, carry0)
        ks = res[:TOPK]
        ix = res[TOPK:]
        wtok = tok * TOPK
        itok = tok * N_OUT
        for j in range(TOPK):
            bj = plsc.load_gather(bias_v, [ix[j]])
            val = (ks[j] - bj) * ROUTE_SCALE
            plsc.store_scatter(w_v, [wtok + j], val)
            plsc.store_scatter(i_v, [itok + (j + N_SHARED)], ix[j] + N_SHARED)
        for j in range(N_SHARED):
            plsc.store_scatter(i_v, [itok + j], jnp.full((_L,), j, jnp.int32))
        return gcarry

    lax.fori_loop(0, NGRP, group_body, 0)
    pltpu.sync_copy(w_v, w_hbm.at[pl.ds(wid * CHUNK * TOPK, CHUNK * TOPK)])
    pltpu.sync_copy(i_v, i_hbm.at[pl.ds(wid * CHUNK * N_OUT, CHUNK * N_OUT)])


@functools.cache
def _topk_sc_fn():
    mesh = plsc.VectorSubcoreMesh(core_axis_name="c", subcore_axis_name="s")
    assert mesh.num_cores * mesh.num_subcores == NW
    body = functools.partial(_topk_body, nc=mesh.num_cores)
    return pl.kernel(
        body,
        mesh=mesh,
        compiler_params=pltpu.CompilerParams(needs_layout_passes=False),
        out_type=[
            jax.ShapeDtypeStruct((CTOK * TOPK,), jnp.float32),
            jax.ShapeDtypeStruct((CTOK * N_OUT,), jnp.int32),
        ],
        scratch_types=[
            pltpu.VMEM((CHUNK * E,), jnp.float32),
            pltpu.VMEM((E,), jnp.float32),
            pltpu.VMEM((CHUNK * TOPK,), jnp.float32),
            pltpu.VMEM((CHUNK * N_OUT,), jnp.int32),
        ],
    )


def kernel(x, W, gate_bias):
    x2d = x.reshape(N_TOK, D)
    wt = W.T
    bias2d = gate_bias.reshape(1, E)
    topk = _topk_sc_fn()
    w_parts, i_parts = [], []
    for c in range(NCHUNK):
        keys = _keys(x2d, wt, bias2d, CTOK, c * (CTOK // TBLK))
        w_c, i_c = topk(keys.reshape(-1), gate_bias)
        w_parts.append(w_c)
        i_parts.append(i_c)
    weights = jnp.concatenate(w_parts)
    indices = jnp.concatenate(i_parts)
    return (
        weights.reshape(B, T, TOPK),
        indices.reshape(B, T, N_OUT),
    )


# speedup vs baseline: 1.0472x; 1.0472x over previous
import functools

import jax
import jax.numpy as jnp
from jax import lax
from jax.experimental import pallas as pl
from jax.experimental.pallas import tpu as pltpu
from jax.experimental.pallas import tpu_sc as plsc

B, T, D = 4, 4096, 4096
E = 64
TOPK = 8
N_SHARED = 2
N_OUT = N_SHARED + TOPK
ROUTE_SCALE = 2.5
N_TOK = B * T

TBLK = 512


def _keys_block(x_ref, wt_ref, bias_ref, k_ref):
    s = jnp.dot(x_ref[...], wt_ref[...], preferred_element_type=jnp.float32)
    m = jnp.max(s, axis=1, keepdims=True)
    p = jnp.exp(s - m)
    probs = p / jnp.sum(p, axis=1, keepdims=True)
    keys = probs + bias_ref[...]
    k_ref[...] = jnp.concatenate([keys, keys], axis=1)


def _keys(x2d, wt, bias2d, n, blk0):
    return pl.pallas_call(
        _keys_block,
        grid=(n // TBLK,),
        in_specs=[
            pl.BlockSpec((TBLK, D), lambda i: (i + blk0, 0)),
            pl.BlockSpec((D, E), lambda i: (0, 0)),
            pl.BlockSpec((1, E), lambda i: (0, 0)),
        ],
        out_specs=pl.BlockSpec((TBLK, 2 * E), lambda i: (i, 0)),
        out_shape=jax.ShapeDtypeStruct((n, 2 * E), jnp.float32),
    )(x2d, wt, bias2d)


_L = 16
NW = 32
NCHUNK = 4
CTOK = N_TOK // NCHUNK
CHUNK = CTOK // NW
NGRP = CHUNK // _L


def _topk_body(keys_hbm, bias_hbm, w_hbm, i_hbm, keys_v, bias_v, w_v, i_v, nc):
    wid = lax.axis_index("s") * nc + lax.axis_index("c")
    pltpu.sync_copy(keys_hbm.at[pl.ds(wid * CHUNK * 2 * E, CHUNK * 2 * E)], keys_v)
    pltpu.sync_copy(bias_hbm, bias_v)

    def group_body(g, gcarry):
        tok = lax.iota(jnp.int32, _L) + g * _L
        flat0 = tok * (2 * E)
        neg = jnp.full((_L,), -jnp.inf, jnp.float32)
        zero = jnp.zeros((_L,), jnp.int32)
        carry0 = (neg,) * TOPK + (zero,) * TOPK

        def exp_body(e, carry):
            ks = carry[:TOPK]
            ix = carry[TOPK:]
            e_spl = jnp.full((_L,), e, jnp.int32)
            v = plsc.load_gather(keys_v, [flat0 + e])
            c = [v > ks[j] for j in range(TOPK)]
            nk = [jnp.where(c[0], v, ks[0])]
            ni = [jnp.where(c[0], e_spl, ix[0])]
            for j in range(1, TOPK):
                nk.append(jnp.where(c[j], jnp.where(c[j - 1], ks[j - 1], v), ks[j]))
                ni.append(jnp.where(c[j], jnp.where(c[j - 1], ix[j - 1], e_spl), ix[j]))
            return tuple(nk) + tuple(ni)

        res = lax.fori_loop(0, E, exp_body, carry0)
        ks = res[:TOPK]
        ix = res[TOPK:]
        wtok = tok * TOPK
        itok = tok * N_OUT
        for j in range(TOPK):
            bj = plsc.load_gather(bias_v, [ix[j]])
            val = (ks[j] - bj) * ROUTE_SCALE
            plsc.store_scatter(w_v, [wtok + j], val)
            plsc.store_scatter(i_v, [itok + (j + N_SHARED)], ix[j] + N_SHARED)
        for j in range(N_SHARED):
            plsc.store_scatter(i_v, [itok + j], jnp.full((_L,), j, jnp.int32))
        return gcarry

    lax.fori_loop(0, NGRP, group_body, 0)
    pltpu.sync_copy(w_v, w_hbm.at[pl.ds(wid * CHUNK * TOPK, CHUNK * TOPK)])
    pltpu.sync_copy(i_v, i_hbm.at[pl.ds(wid * CHUNK * N_OUT, CHUNK * N_OUT)])


@functools.cache
def _topk_sc_fn():
    mesh = plsc.VectorSubcoreMesh(core_axis_name="c", subcore_axis_name="s")
    assert mesh.num_cores * mesh.num_subcores == NW
    body = functools.partial(_topk_body, nc=mesh.num_cores)
    return pl.kernel(
        body,
        mesh=mesh,
        compiler_params=pltpu.CompilerParams(needs_layout_passes=False),
        out_type=[
            jax.ShapeDtypeStruct((CTOK * TOPK,), jnp.float32),
            jax.ShapeDtypeStruct((CTOK * N_OUT,), jnp.int32),
        ],
        scratch_types=[
            pltpu.VMEM((CHUNK * 2 * E,), jnp.float32),
            pltpu.VMEM((E,), jnp.float32),
            pltpu.VMEM((CHUNK * TOPK,), jnp.float32),
            pltpu.VMEM((CHUNK * N_OUT,), jnp.int32),
        ],
    )


def kernel(x, W, gate_bias):
    x2d = x.reshape(N_TOK, D)
    wt = W.T
    bias2d = gate_bias.reshape(1, E)
    topk = _topk_sc_fn()
    w_parts, i_parts = [], []
    for c in range(NCHUNK):
        keys = _keys(x2d, wt, bias2d, CTOK, c * (CTOK // TBLK))
        w_c, i_c = topk(keys.reshape(-1), gate_bias)
        w_parts.append(w_c)
        i_parts.append(i_c)
    weights = jnp.concatenate(w_parts)
    indices = jnp.concatenate(i_parts)
    return (
        weights.reshape(B, T, TOPK),
        indices.reshape(B, T, N_OUT),
    )


# speedup vs baseline: 1.0817x; 1.0329x over previous
import functools

import jax
import jax.numpy as jnp
from jax import lax
from jax.experimental import pallas as pl
from jax.experimental.pallas import tpu as pltpu
from jax.experimental.pallas import tpu_sc as plsc

B, T, D = 4, 4096, 4096
E = 64
TOPK = 8
N_SHARED = 2
N_OUT = N_SHARED + TOPK
ROUTE_SCALE = 2.5
N_TOK = B * T

TBLK = 512


def _keys_block(x_ref, w_ref, bias_ref, k_ref):
    s = lax.dot_general(
        x_ref[...], w_ref[...], (((1,), (1,)), ((), ())),
        preferred_element_type=jnp.float32,
    )
    m = jnp.max(s, axis=1, keepdims=True)
    p = jnp.exp(s - m)
    probs = p / jnp.sum(p, axis=1, keepdims=True)
    keys = probs + bias_ref[...]
    k_ref[...] = jnp.concatenate([keys, keys], axis=1)


def _keys(x2d, w, bias2d, n, blk0):
    return pl.pallas_call(
        _keys_block,
        grid=(n // TBLK,),
        in_specs=[
            pl.BlockSpec((TBLK, D), lambda i: (i + blk0, 0)),
            pl.BlockSpec((E, D), lambda i: (0, 0)),
            pl.BlockSpec((1, E), lambda i: (0, 0)),
        ],
        out_specs=pl.BlockSpec((TBLK, 2 * E), lambda i: (i, 0)),
        out_shape=jax.ShapeDtypeStruct((n, 2 * E), jnp.float32),
    )(x2d, w, bias2d)


_L = 16
NW = 32
NCHUNK = 4
CTOK = N_TOK // NCHUNK
CHUNK = CTOK // NW
NGRP = CHUNK // _L


def _topk_body(keys_hbm, bias_hbm, w_hbm, i_hbm, keys_v, bias_v, w_v, i_v, nc):
    wid = lax.axis_index("s") * nc + lax.axis_index("c")
    pltpu.sync_copy(keys_hbm.at[pl.ds(wid * CHUNK * 2 * E, CHUNK * 2 * E)], keys_v)
    pltpu.sync_copy(bias_hbm, bias_v)

    def group_body(g, gcarry):
        tok = lax.iota(jnp.int32, _L) + g * _L
        flat0 = tok * (2 * E)
        neg = jnp.full((_L,), -jnp.inf, jnp.float32)
        zero = jnp.zeros((_L,), jnp.int32)
        carry0 = (neg,) * TOPK + (zero,) * TOPK

        def exp_body(e, carry):
            ks = carry[:TOPK]
            ix = carry[TOPK:]
            e_spl = jnp.full((_L,), e, jnp.int32)
            v = plsc.load_gather(keys_v, [flat0 + e])
            c = [v > ks[j] for j in range(TOPK)]
            nk = [jnp.where(c[0], v, ks[0])]
            ni = [jnp.where(c[0], e_spl, ix[0])]
            for j in range(1, TOPK):
                nk.append(jnp.where(c[j], jnp.where(c[j - 1], ks[j - 1], v), ks[j]))
                ni.append(jnp.where(c[j], jnp.where(c[j - 1], ix[j - 1], e_spl), ix[j]))
            return tuple(nk) + tuple(ni)

        res = lax.fori_loop(0, E, exp_body, carry0)
        ks = res[:TOPK]
        ix = res[TOPK:]
        wtok = tok * TOPK
        itok = tok * N_OUT
        for j in range(TOPK):
            bj = plsc.load_gather(bias_v, [ix[j]])
            val = (ks[j] - bj) * ROUTE_SCALE
            plsc.store_scatter(w_v, [wtok + j], val)
            plsc.store_scatter(i_v, [itok + (j + N_SHARED)], ix[j] + N_SHARED)
        for j in range(N_SHARED):
            plsc.store_scatter(i_v, [itok + j], jnp.full((_L,), j, jnp.int32))
        return gcarry

    lax.fori_loop(0, NGRP, group_body, 0)
    pltpu.sync_copy(w_v, w_hbm.at[pl.ds(wid * CHUNK * TOPK, CHUNK * TOPK)])
    pltpu.sync_copy(i_v, i_hbm.at[pl.ds(wid * CHUNK * N_OUT, CHUNK * N_OUT)])


@functools.cache
def _topk_sc_fn():
    mesh = plsc.VectorSubcoreMesh(core_axis_name="c", subcore_axis_name="s")
    assert mesh.num_cores * mesh.num_subcores == NW
    body = functools.partial(_topk_body, nc=mesh.num_cores)
    return pl.kernel(
        body,
        mesh=mesh,
        compiler_params=pltpu.CompilerParams(needs_layout_passes=False),
        out_type=[
            jax.ShapeDtypeStruct((CTOK * TOPK,), jnp.float32),
            jax.ShapeDtypeStruct((CTOK * N_OUT,), jnp.int32),
        ],
        scratch_types=[
            pltpu.VMEM((CHUNK * 2 * E,), jnp.float32),
            pltpu.VMEM((E,), jnp.float32),
            pltpu.VMEM((CHUNK * TOPK,), jnp.float32),
            pltpu.VMEM((CHUNK * N_OUT,), jnp.int32),
        ],
    )


def kernel(x, W, gate_bias):
    x2d = x.reshape(N_TOK, D)
    bias2d = gate_bias.reshape(1, E)
    topk = _topk_sc_fn()
    w_parts, i_parts = [], []
    for c in range(NCHUNK):
        keys = _keys(x2d, W, bias2d, CTOK, c * (CTOK // TBLK))
        w_c, i_c = topk(keys.reshape(-1), gate_bias)
        w_parts.append(w_c)
        i_parts.append(i_c)
    weights = jnp.concatenate(w_parts)
    indices = jnp.concatenate(i_parts)
    return (
        weights.reshape(B, T, TOPK),
        indices.reshape(B, T, N_OUT),
    )
